# Initial kernel scaffold; baseline (speedup 1.0000x reference)
import jax
import jax.numpy as jnp
from jax.experimental import pallas as pl


def kernel(x_node, x_edge, edge_index, eb_W1, eb_b1, eb_W2, eb_b2, eb_W3, eb_b3, eb_W4, eb_b4, eb_g, eb_beta, nb_W1, nb_b1, nb_W2, nb_b2, nb_W3, nb_b3, nb_W4, nb_b4, nb_g, nb_beta):
    raise NotImplementedError("write your pallas kernel here")


# speedup vs baseline: 3.9018x; 3.9018x over previous
import functools

import jax
import jax.numpy as jnp
from jax import lax
from jax.experimental import pallas as pl
from jax.experimental.pallas import tpu as pltpu
from jax.experimental.pallas import tpu_sc as plsc

H = 32
LANES = 16
_SC_PARAMS = pltpu.CompilerParams(use_tc_tiling_on_sc=False)


def _pq_tables(x4, wa4, wb4, block_rows):
    rows, width = x4.shape
    grid = (rows // block_rows,)

    def body(x_ref, wa_ref, wb_ref, p_ref, q_ref):
        x = x_ref[...]
        p_ref[...] = jnp.dot(x, wa_ref[...], preferred_element_type=jnp.float32)
        q_ref[...] = jnp.dot(x, wb_ref[...], preferred_element_type=jnp.float32)

    data = pl.BlockSpec((block_rows, width), lambda i: (i, 0))
    full = pl.BlockSpec((width, width), lambda i: (0, 0))
    out = jax.ShapeDtypeStruct((rows, width), jnp.float32)
    return pl.pallas_call(
        body,
        grid=grid,
        in_specs=[data, full, full],
        out_specs=[data, data],
        out_shape=[out, out],
    )(x4, wa4, wb4)


def _edge_mlp(g2, xe2, w1c, b1, w2, b2, w3, b3, w4, b4, gv, bv, mm, wyb,
              block_rows):
    rows, width = g2.shape
    grid = (rows // block_rows,)

    def body(g_ref, x_ref, w1c_r, b1_r, w2_r, b2_r, w3_r, b3_r, w4_r, b4_r,
             gv_r, bv_r, mm_r, wyb_r, o_ref, y_ref):
        f32 = jnp.float32
        xe = x_ref[...]
        h = g_ref[...] + jnp.dot(xe, w1c_r[...], preferred_element_type=f32)
        h = jnp.maximum(h + b1_r[...], 0.0)
        h = jnp.maximum(jnp.dot(h, w2_r[...], preferred_element_type=f32) + b2_r[...], 0.0)
        h = jnp.maximum(jnp.dot(h, w3_r[...], preferred_element_type=f32) + b3_r[...], 0.0)
        h = jnp.dot(h, w4_r[...], preferred_element_type=f32) + b4_r[...]
        mu = jnp.dot(h, mm_r[...], preferred_element_type=f32)
        hc = h - mu
        var = jnp.dot(hc * hc, mm_r[...], preferred_element_type=f32)
        xeo = hc * lax.rsqrt(var + 1e-5) * gv_r[...] + bv_r[...]
        o_ref[...] = xe + xeo
        y_ref[...] = jnp.dot(xeo, wyb_r[...], preferred_element_type=f32)

    data = pl.BlockSpec((block_rows, width), lambda i: (i, 0))
    mat = pl.BlockSpec((width, width), lambda i: (0, 0))
    vec = pl.BlockSpec((1, width), lambda i: (0, 0))
    out = jax.ShapeDtypeStruct((rows, width), jnp.float32)
    return pl.pallas_call(
        body,
        grid=grid,
        in_specs=[data, data, mat, vec, mat, vec, mat, vec, mat, vec,
                  vec, vec, mat, mat],
        out_specs=[data, data],
        out_shape=[out, out],
    )(g2, xe2, w1c, b1, w2, b2, w3, b3, w4, b4, gv, bv, mm, wyb)


def _node_mlp(x4, s4, w1a, b1, w2, b2, w3, b3, w4, b4, gv, bv, mm, block_rows):
    rows, width = x4.shape
    grid = (rows // block_rows,)

    def body(x_ref, s_ref, w1a_r, b1_r, w2_r, b2_r, w3_r, b3_r, w4_r, b4_r,
             gv_r, bv_r, mm_r, o_ref):
        f32 = jnp.float32
        x = x_ref[...]
        h = jnp.dot(x, w1a_r[...], preferred_element_type=f32) + s_ref[...]
        h = jnp.maximum(h + b1_r[...], 0.0)
        h = jnp.maximum(jnp.dot(h, w2_r[...], preferred_element_type=f32) + b2_r[...], 0.0)
        h = jnp.maximum(jnp.dot(h, w3_r[...], preferred_element_type=f32) + b3_r[...], 0.0)
        h = jnp.dot(h, w4_r[...], preferred_element_type=f32) + b4_r[...]
        mu = jnp.dot(h, mm_r[...], preferred_element_type=f32)
        hc = h - mu
        var = jnp.dot(hc * hc, mm_r[...], preferred_element_type=f32)
        o_ref[...] = x + hc * lax.rsqrt(var + 1e-5) * gv_r[...] + bv_r[...]

    data = pl.BlockSpec((block_rows, width), lambda i: (i, 0))
    mat = pl.BlockSpec((width, width), lambda i: (0, 0))
    vec = pl.BlockSpec((1, width), lambda i: (0, 0))
    out = jax.ShapeDtypeStruct((rows, width), jnp.float32)
    return pl.pallas_call(
        body,
        grid=grid,
        in_specs=[data, data, mat, vec, mat, vec, mat, vec, mat, vec,
                  vec, vec, mat],
        out_specs=data,
        out_shape=out,
    )(x4, s4, w1a, b1, w2, b2, w3, b3, w4, b4, gv, bv, mm)


def _sc_gather_sum(p, q, e0, e1):
    n, h = p.shape
    e = e0.shape[0]
    info = plsc.get_sparse_core_info()
    nw = info.num_cores * info.num_subcores
    epw = e // nw
    ch = 128
    nfull = epw // ch
    tail = epw - nfull * ch
    mesh = plsc.VectorSubcoreMesh(core_axis_name="c", subcore_axis_name="s")

    scratch = [
        pltpu.VMEM((ch,), jnp.int32),
        pltpu.VMEM((ch,), jnp.int32),
        pltpu.VMEM((ch, h), jnp.float32),
        pltpu.VMEM((ch, h), jnp.float32),
        pltpu.SemaphoreType.DMA,
        pltpu.SemaphoreType.DMA,
    ]
    if tail:
        scratch += [
            pltpu.VMEM((tail,), jnp.int32),
            pltpu.VMEM((tail,), jnp.int32),
            pltpu.VMEM((tail, h), jnp.float32),
            pltpu.VMEM((tail, h), jnp.float32),
        ]

    @functools.partial(
        pl.kernel,
        out_type=jax.ShapeDtypeStruct((e, h), jnp.float32),
        mesh=mesh,
        scratch_types=scratch,
        compiler_params=_SC_PARAMS,
    )
    def k(p_hbm, q_hbm, e0_hbm, e1_hbm, g_hbm, *scr):
        if tail:
            i0, i1, ba, bb, s0, s1, i0t, i1t, bat, bbt = scr
        else:
            i0, i1, ba, bb, s0, s1 = scr
        wid = lax.axis_index("s") * info.num_cores + lax.axis_index("c")
        base = wid * epw

        def process(off, nrows, ia, ib, da, db):
            pltpu.sync_copy(e0_hbm.at[pl.ds(off, nrows)], ia)
            pltpu.sync_copy(e1_hbm.at[pl.ds(off, nrows)], ib)
            ca = pltpu.async_copy(p_hbm.at[ia], da, s0)
            cb = pltpu.async_copy(q_hbm.at[ib], db, s1)
            ca.wait()
            cb.wait()

            @pl.loop(0, nrows)
            def _(r):
                for half in range(h // LANES):
                    slc = (pl.ds(r, 1), pl.ds(half * LANES, LANES))
                    da.at[*slc][...] = da.at[*slc][...] + db.at[*slc][...]

            pltpu.sync_copy(da, g_hbm.at[pl.ds(off, nrows)])

        @pl.loop(0, nfull)
        def _(ci):
            process(base + ci * ch, ch, i0, i1, ba, bb)

        if tail:
            process(base + nfull * ch, tail, i0t, i1t, bat, bbt)

    return k(p, q, e0, e1)


def _sc_scatter_add(ye, e0, e1, n):
    e, h = ye.shape
    info = plsc.get_sparse_core_info()
    nc, ns = info.num_cores, info.num_subcores
    hn = n // nc
    acc_rows = hn + ns
    eps = e // ns
    ch = 128
    nfull = eps // ch
    tail = eps - nfull * ch
    zr = acc_rows // ns
    ops = hn // ns
    mesh = plsc.VectorSubcoreMesh(core_axis_name="c", subcore_axis_name="s")

    scratch = [
        pltpu.VMEM((ch, h), jnp.float32),
        pltpu.VMEM((ch,), jnp.int32),
        pltpu.VMEM((ch,), jnp.int32),
        pltpu.VMEM_SHARED((acc_rows, h), jnp.float32),
    ]
    if tail:
        scratch += [
            pltpu.VMEM((tail, h), jnp.float32),
            pltpu.VMEM((tail,), jnp.int32),
            pltpu.VMEM((tail,), jnp.int32),
        ]

    zrows = jnp.zeros((zr, h), jnp.float32)

    @functools.partial(
        pl.kernel,
        out_type=jax.ShapeDtypeStruct((n, h), jnp.float32),
        mesh=mesh,
        scratch_types=scratch,
        compiler_params=_SC_PARAMS,
    )
    def k(ye_hbm, e0_hbm, e1_hbm, z_hbm, s_hbm, *scr):
        if tail:
            yb, i0, i1, acc, ybt, i0t, i1t = scr
        else:
            yb, i0, i1, acc = scr
        c = lax.axis_index("c")
        sid = lax.axis_index("s")
        nbase = c * hn
        dummy = hn + sid

        pltpu.sync_copy(z_hbm, acc.at[pl.ds(sid * zr, zr)])
        plsc.subcore_barrier()

        ebase = sid * eps

        def process(off, nrows, buf, ia, ib):
            pltpu.sync_copy(ye_hbm.at[pl.ds(off, nrows)], buf)
            pltpu.sync_copy(e0_hbm.at[pl.ds(off, nrows)], ia)
            pltpu.sync_copy(e1_hbm.at[pl.ds(off, nrows)], ib)
            for iref in (ia, ib):
                @pl.loop(0, nrows // LANES)
                def _(kk):
                    s = pl.ds(kk * LANES, LANES)
                    v = iref.at[s][...]
                    inb = (v >= nbase) & (v < nbase + hn)
                    iref.at[s][...] = jnp.where(inb, v - nbase, dummy)
            pltpu.sync_copy(buf, acc.at[ia], add=True)
            pltpu.sync_copy(buf, acc.at[ib], add=True)

        @pl.loop(0, nfull)
        def _(ci):
            process(ebase + ci * ch, ch, yb, i0, i1)

        if tail:
            process(ebase + nfull * ch, tail, ybt, i0t, i1t)

        plsc.subcore_barrier()
        pltpu.sync_copy(acc.at[pl.ds(sid * ops, ops)],
                        s_hbm.at[pl.ds(nbase + sid * ops, ops)])

    return k(ye, e0, e1, zrows)


def _bd(w, f):
    return jnp.kron(jnp.eye(f, dtype=w.dtype), w)


def _tile(v, f):
    return jnp.tile(v, f).reshape(1, f * v.shape[0])


def kernel(x_node, x_edge, edge_index, eb_W1, eb_b1, eb_W2, eb_b2, eb_W3,
           eb_b3, eb_W4, eb_b4, eb_g, eb_beta, nb_W1, nb_b1, nb_W2, nb_b2,
           nb_W3, nb_b3, nb_W4, nb_b4, nb_g, nb_beta):
    n = x_node.shape[0]
    e = x_edge.shape[0]
    e0 = edge_index[0]
    e1 = edge_index[1]

    fn, fe = 4, 8
    wn, we = fn * H, fe * H
    x4 = x_node.reshape(n // fn, wn)

    w1a4 = _bd(eb_W1[:H], fn)
    w1b4 = _bd(eb_W1[H:2 * H], fn)
    w1c8 = _bd(eb_W1[2 * H:], fe)
    eb_mats = [_bd(w, fe) for w in (eb_W2, eb_W3, eb_W4)]
    eb_vecs = [_tile(v, fe) for v in (eb_b1, eb_b2, eb_b3, eb_b4, eb_g, eb_beta)]
    mean8 = _bd(jnp.full((H, H), 1.0 / H, jnp.float32), fe)
    wyb8 = _bd(nb_W1[H:], fe)

    nb_mats = [_bd(w, fn) for w in (nb_W1[:H], nb_W2, nb_W3, nb_W4)]
    nb_vecs = [_tile(v, fn) for v in (nb_b1, nb_b2, nb_b3, nb_b4, nb_g, nb_beta)]
    mean4 = _bd(jnp.full((H, H), 1.0 / H, jnp.float32), fn)

    p4, q4 = _pq_tables(x4, w1a4, w1b4, block_rows=1000)
    p = p4.reshape(n, H)
    q = q4.reshape(n, H)

    g = _sc_gather_sum(p, q, e0, e1)

    out_e8, ye8 = _edge_mlp(
        g.reshape(e // fe, we), x_edge.reshape(e // fe, we),
        w1c8, eb_vecs[0], eb_mats[0], eb_vecs[1], eb_mats[1], eb_vecs[2],
        eb_mats[2], eb_vecs[3], eb_vecs[4], eb_vecs[5], mean8, wyb8,
        block_rows=1000)

    s = _sc_scatter_add(ye8.reshape(e, H), e0, e1, n)

    out_n4 = _node_mlp(
        x4, s.reshape(n // fn, wn),
        nb_mats[0], nb_vecs[0], nb_mats[1], nb_vecs[1], nb_mats[2],
        nb_vecs[2], nb_mats[3], nb_vecs[3], nb_vecs[4], nb_vecs[5], mean4,
        block_rows=1000)

    return out_n4.reshape(n, H), out_e8.reshape(e, H)


# speedup vs baseline: 6.1032x; 1.5642x over previous
import functools

import jax
import jax.numpy as jnp
from jax import lax
from jax.experimental import pallas as pl
from jax.experimental.pallas import tpu as pltpu
from jax.experimental.pallas import tpu_sc as plsc

H = 32
LANES = 16
_SC_PARAMS = pltpu.CompilerParams(use_tc_tiling_on_sc=False)


def _pq_tables(x4, wa4, wb4, block_rows):
    rows, width = x4.shape
    grid = (rows // block_rows,)

    def body(x_ref, wa_ref, wb_ref, p_ref, q_ref):
        x = x_ref[...]
        p_ref[...] = jnp.dot(x, wa_ref[...], preferred_element_type=jnp.float32)
        q_ref[...] = jnp.dot(x, wb_ref[...], preferred_element_type=jnp.float32)

    data = pl.BlockSpec((block_rows, width), lambda i: (i, 0))
    full = pl.BlockSpec((width, width), lambda i: (0, 0))
    out = jax.ShapeDtypeStruct((rows, width), jnp.float32)
    return pl.pallas_call(
        body,
        grid=grid,
        in_specs=[data, full, full],
        out_specs=[data, data],
        out_shape=[out, out],
    )(x4, wa4, wb4)


def _edge_mlp(g2, xe2, w1c, b1, w2, b2, w3, b3, w4, b4, gv, bv, mm, wyb,
              block_rows):
    rows, width = g2.shape
    grid = (rows // block_rows,)

    def body(g_ref, x_ref, w1c_r, b1_r, w2_r, b2_r, w3_r, b3_r, w4_r, b4_r,
             gv_r, bv_r, mm_r, wyb_r, o_ref, y_ref):
        f32 = jnp.float32
        xe = x_ref[...]
        h = g_ref[...] + jnp.dot(xe, w1c_r[...], preferred_element_type=f32)
        h = jnp.maximum(h + b1_r[...], 0.0)
        h = jnp.maximum(jnp.dot(h, w2_r[...], preferred_element_type=f32) + b2_r[...], 0.0)
        h = jnp.maximum(jnp.dot(h, w3_r[...], preferred_element_type=f32) + b3_r[...], 0.0)
        h = jnp.dot(h, w4_r[...], preferred_element_type=f32) + b4_r[...]
        mu = jnp.dot(h, mm_r[...], preferred_element_type=f32)
        hc = h - mu
        var = jnp.dot(hc * hc, mm_r[...], preferred_element_type=f32)
        xeo = hc * lax.rsqrt(var + 1e-5) * gv_r[...] + bv_r[...]
        o_ref[...] = xe + xeo
        y_ref[...] = jnp.dot(xeo, wyb_r[...], preferred_element_type=f32)

    data = pl.BlockSpec((block_rows, width), lambda i: (i, 0))
    mat = pl.BlockSpec((width, width), lambda i: (0, 0))
    vec = pl.BlockSpec((1, width), lambda i: (0, 0))
    out = jax.ShapeDtypeStruct((rows, width), jnp.float32)
    return pl.pallas_call(
        body,
        grid=grid,
        in_specs=[data, data, mat, vec, mat, vec, mat, vec, mat, vec,
                  vec, vec, mat, mat],
        out_specs=[data, data],
        out_shape=[out, out],
    )(g2, xe2, w1c, b1, w2, b2, w3, b3, w4, b4, gv, bv, mm, wyb)


def _node_mlp(x4, s4, w1a, b1, w2, b2, w3, b3, w4, b4, gv, bv, mm, block_rows):
    rows, width = x4.shape
    grid = (rows // block_rows,)

    def body(x_ref, s_ref, w1a_r, b1_r, w2_r, b2_r, w3_r, b3_r, w4_r, b4_r,
             gv_r, bv_r, mm_r, o_ref):
        f32 = jnp.float32
        x = x_ref[...]
        h = jnp.dot(x, w1a_r[...], preferred_element_type=f32) + s_ref[...]
        h = jnp.maximum(h + b1_r[...], 0.0)
        h = jnp.maximum(jnp.dot(h, w2_r[...], preferred_element_type=f32) + b2_r[...], 0.0)
        h = jnp.maximum(jnp.dot(h, w3_r[...], preferred_element_type=f32) + b3_r[...], 0.0)
        h = jnp.dot(h, w4_r[...], preferred_element_type=f32) + b4_r[...]
        mu = jnp.dot(h, mm_r[...], preferred_element_type=f32)
        hc = h - mu
        var = jnp.dot(hc * hc, mm_r[...], preferred_element_type=f32)
        o_ref[...] = x + hc * lax.rsqrt(var + 1e-5) * gv_r[...] + bv_r[...]

    data = pl.BlockSpec((block_rows, width), lambda i: (i, 0))
    mat = pl.BlockSpec((width, width), lambda i: (0, 0))
    vec = pl.BlockSpec((1, width), lambda i: (0, 0))
    out = jax.ShapeDtypeStruct((rows, width), jnp.float32)
    return pl.pallas_call(
        body,
        grid=grid,
        in_specs=[data, data, mat, vec, mat, vec, mat, vec, mat, vec,
                  vec, vec, mat],
        out_specs=data,
        out_shape=out,
    )(x4, s4, w1a, b1, w2, b2, w3, b3, w4, b4, gv, bv, mm)


def _sc_gather_sum(p, q, e0, e1):
    n, h = p.shape
    e = e0.shape[0]
    info = plsc.get_sparse_core_info()
    nw = info.num_cores * info.num_subcores
    epw = e // nw
    ch = 200
    nwin = epw // ch
    assert epw % ch == 0 and nwin % 2 == 0
    mesh = plsc.VectorSubcoreMesh(core_axis_name="c", subcore_axis_name="s")

    scratch = []
    for _ in range(2):
        scratch += [
            pltpu.VMEM((ch,), jnp.int32),
            pltpu.VMEM((ch,), jnp.int32),
            pltpu.VMEM((ch, h), jnp.float32),
            pltpu.VMEM((ch, h), jnp.float32),
            pltpu.VMEM((ch, h), jnp.float32),
            pltpu.SemaphoreType.DMA,
            pltpu.SemaphoreType.DMA,
        ]

    @functools.partial(
        pl.kernel,
        out_type=jax.ShapeDtypeStruct((e, h), jnp.float32),
        mesh=mesh,
        scratch_types=scratch,
        compiler_params=_SC_PARAMS,
    )
    def k(p_hbm, q_hbm, e0_hbm, e1_hbm, g_hbm, *scr):
        i0 = [scr[0], scr[7]]
        i1 = [scr[1], scr[8]]
        ba = [scr[2], scr[9]]
        bb = [scr[3], scr[10]]
        do = [scr[4], scr[11]]
        gs = [scr[5], scr[12]]
        st = [scr[6], scr[13]]
        wid = lax.axis_index("s") * info.num_cores + lax.axis_index("c")
        base = wid * epw

        def fetch(b, off):
            pltpu.sync_copy(e0_hbm.at[pl.ds(off, ch)], i0[b])
            pltpu.sync_copy(e1_hbm.at[pl.ds(off, ch)], i1[b])
            pltpu.async_copy(p_hbm.at[i0[b]], ba[b], gs[b])
            pltpu.async_copy(q_hbm.at[i1[b]], bb[b], gs[b])

        for b in range(2):
            fetch(b, base + b * ch)

        @pl.loop(0, nwin // 2)
        def _(gi):
            for b in range(2):
                off = base + (gi * 2 + b) * ch
                pltpu.make_async_copy(p_hbm.at[i0[b]], ba[b], gs[b]).wait()
                pltpu.make_async_copy(q_hbm.at[i1[b]], bb[b], gs[b]).wait()

                @pl.when(gi > 0)
                def _():
                    pltpu.make_async_copy(
                        do[b], g_hbm.at[pl.ds(off - 2 * ch, ch)], st[b]).wait()

                @pl.loop(0, ch)
                def _(r):
                    for half in range(h // LANES):
                        slc = (pl.ds(r, 1), pl.ds(half * LANES, LANES))
                        do[b].at[*slc][...] = (
                            ba[b].at[*slc][...] + bb[b].at[*slc][...])

                pltpu.async_copy(do[b], g_hbm.at[pl.ds(off, ch)], st[b])

                @pl.when(gi * 2 + b + 2 < nwin)
                def _():
                    fetch(b, off + 2 * ch)

        for b in range(2):
            pltpu.make_async_copy(
                do[b],
                g_hbm.at[pl.ds(base + (nwin - 2 + b) * ch, ch)],
                st[b]).wait()

    return k(p, q, e0, e1)


def _sc_scatter_add(ye, e0, e1, n):
    e, h = ye.shape
    info = plsc.get_sparse_core_info()
    nc, ns = info.num_cores, info.num_subcores
    hn = n // nc
    acc_rows = hn + ns
    eps = e // ns
    ch = 400
    nwin = eps // ch
    assert eps % ch == 0 and nwin % 2 == 0 and ch % LANES == 0
    zr = acc_rows // ns
    ops = hn // ns
    mesh = plsc.VectorSubcoreMesh(core_axis_name="c", subcore_axis_name="s")

    scratch = [pltpu.VMEM_SHARED((acc_rows, h), jnp.float32)]
    for _ in range(2):
        scratch += [
            pltpu.VMEM((ch, h), jnp.float32),
            pltpu.VMEM((ch,), jnp.int32),
            pltpu.VMEM((ch,), jnp.int32),
            pltpu.SemaphoreType.DMA,
        ]

    zrows = jnp.zeros((zr, h), jnp.float32)

    @functools.partial(
        pl.kernel,
        out_type=jax.ShapeDtypeStruct((n, h), jnp.float32),
        mesh=mesh,
        scratch_types=scratch,
        compiler_params=_SC_PARAMS,
    )
    def k(ye_hbm, e0_hbm, e1_hbm, z_hbm, s_hbm, *scr):
        acc = scr[0]
        yb = [scr[1], scr[5]]
        i0 = [scr[2], scr[6]]
        i1 = [scr[3], scr[7]]
        sc = [scr[4], scr[8]]
        c = lax.axis_index("c")
        sid = lax.axis_index("s")
        nbase = c * hn
        dummy = hn + sid

        pltpu.sync_copy(z_hbm, acc.at[pl.ds(sid * zr, zr)])
        plsc.subcore_barrier()

        ebase = sid * eps

        def load(b, off):
            pltpu.sync_copy(ye_hbm.at[pl.ds(off, ch)], yb[b])
            pltpu.sync_copy(e0_hbm.at[pl.ds(off, ch)], i0[b])
            pltpu.sync_copy(e1_hbm.at[pl.ds(off, ch)], i1[b])
            for iref in (i0[b], i1[b]):
                @pl.loop(0, ch // LANES)
                def _(kk):
                    s = pl.ds(kk * LANES, LANES)
                    v = iref.at[s][...]
                    inb = (v >= nbase) & (v < nbase + hn)
                    iref.at[s][...] = jnp.where(inb, v - nbase, dummy)

        def drain(b):
            pltpu.make_async_copy(yb[b], acc.at[i0[b]], sc[b]).wait()
            pltpu.make_async_copy(yb[b], acc.at[i1[b]], sc[b]).wait()

        load(0, ebase)

        @pl.loop(0, nwin // 2)
        def _(gi):
            for b in range(2):
                w0 = gi * 2 + b
                pltpu.async_copy(yb[b], acc.at[i0[b]], sc[b], add=True)
                pltpu.async_copy(yb[b], acc.at[i1[b]], sc[b], add=True)

                @pl.when(w0 + 1 < nwin)
                def _():
                    b2 = 1 - b

                    @pl.when(w0 >= 1)
                    def _():
                        drain(b2)

                    load(b2, ebase + (w0 + 1) * ch)

        for b in range(2):
            drain(b)

        plsc.subcore_barrier()
        pltpu.sync_copy(acc.at[pl.ds(sid * ops, ops)],
                        s_hbm.at[pl.ds(nbase + sid * ops, ops)])

    return k(ye, e0, e1, zrows)


def _bd(w, f):
    return jnp.kron(jnp.eye(f, dtype=w.dtype), w)


def _tile(v, f):
    return jnp.tile(v, f).reshape(1, f * v.shape[0])


def kernel(x_node, x_edge, edge_index, eb_W1, eb_b1, eb_W2, eb_b2, eb_W3,
           eb_b3, eb_W4, eb_b4, eb_g, eb_beta, nb_W1, nb_b1, nb_W2, nb_b2,
           nb_W3, nb_b3, nb_W4, nb_b4, nb_g, nb_beta):
    n = x_node.shape[0]
    e = x_edge.shape[0]
    e0 = edge_index[0]
    e1 = edge_index[1]

    fn, fe = 4, 8
    wn, we = fn * H, fe * H
    x4 = x_node.reshape(n // fn, wn)

    w1a4 = _bd(eb_W1[:H], fn)
    w1b4 = _bd(eb_W1[H:2 * H], fn)
    w1c8 = _bd(eb_W1[2 * H:], fe)
    eb_mats = [_bd(w, fe) for w in (eb_W2, eb_W3, eb_W4)]
    eb_vecs = [_tile(v, fe) for v in (eb_b1, eb_b2, eb_b3, eb_b4, eb_g, eb_beta)]
    mean8 = _bd(jnp.full((H, H), 1.0 / H, jnp.float32), fe)
    wyb8 = _bd(nb_W1[H:], fe)

    nb_mats = [_bd(w, fn) for w in (nb_W1[:H], nb_W2, nb_W3, nb_W4)]
    nb_vecs = [_tile(v, fn) for v in (nb_b1, nb_b2, nb_b3, nb_b4, nb_g, nb_beta)]
    mean4 = _bd(jnp.full((H, H), 1.0 / H, jnp.float32), fn)

    p4, q4 = _pq_tables(x4, w1a4, w1b4, block_rows=1000)
    p = p4.reshape(n, H)
    q = q4.reshape(n, H)

    g = _sc_gather_sum(p, q, e0, e1)

    out_e8, ye8 = _edge_mlp(
        g.reshape(e // fe, we), x_edge.reshape(e // fe, we),
        w1c8, eb_vecs[0], eb_mats[0], eb_vecs[1], eb_mats[1], eb_vecs[2],
        eb_mats[2], eb_vecs[3], eb_vecs[4], eb_vecs[5], mean8, wyb8,
        block_rows=1000)

    s = _sc_scatter_add(ye8.reshape(e, H), e0, e1, n)

    out_n4 = _node_mlp(
        x4, s.reshape(n // fn, wn),
        nb_mats[0], nb_vecs[0], nb_mats[1], nb_vecs[1], nb_mats[2],
        nb_vecs[2], nb_mats[3], nb_vecs[3], nb_vecs[4], nb_vecs[5], mean4,
        block_rows=1000)

    return out_n4.reshape(n, H), out_e8.reshape(e, H)


# speedup vs baseline: 7.0837x; 1.1607x over previous
import functools

import jax
import jax.numpy as jnp
from jax import lax
from jax.experimental import pallas as pl
from jax.experimental.pallas import tpu as pltpu
from jax.experimental.pallas import tpu_sc as plsc

H = 32
LANES = 16
_SC_PARAMS = pltpu.CompilerParams(use_tc_tiling_on_sc=False)


def _pq_tables(x4, wa4, wb4, block_rows):
    rows, width = x4.shape
    grid = (rows // block_rows,)

    def body(x_ref, wa_ref, wb_ref, p_ref, q_ref):
        x = x_ref[...]
        p_ref[...] = jnp.dot(x, wa_ref[...], preferred_element_type=jnp.float32)
        q_ref[...] = jnp.dot(x, wb_ref[...], preferred_element_type=jnp.float32)

    data = pl.BlockSpec((block_rows, width), lambda i: (i, 0))
    full = pl.BlockSpec((width, width), lambda i: (0, 0))
    out = jax.ShapeDtypeStruct((rows, width), jnp.float32)
    return pl.pallas_call(
        body,
        grid=grid,
        in_specs=[data, full, full],
        out_specs=[data, data],
        out_shape=[out, out],
    )(x4, wa4, wb4)


def _edge_mlp(g2, xe2, w1c, b1, w2, b2, w3, b3, w4, b4, gv, bv, mm, wyb,
              block_rows):
    rows, width = g2.shape
    grid = (rows // block_rows,)

    def body(g_ref, x_ref, w1c_r, b1_r, w2_r, b2_r, w3_r, b3_r, w4_r, b4_r,
             gv_r, bv_r, mm_r, wyb_r, o_ref, y_ref):
        f32 = jnp.float32
        xe = x_ref[...]
        h = g_ref[...] + jnp.dot(xe, w1c_r[...], preferred_element_type=f32)
        h = jnp.maximum(h + b1_r[...], 0.0)
        h = jnp.maximum(jnp.dot(h, w2_r[...], preferred_element_type=f32) + b2_r[...], 0.0)
        h = jnp.maximum(jnp.dot(h, w3_r[...], preferred_element_type=f32) + b3_r[...], 0.0)
        h = jnp.dot(h, w4_r[...], preferred_element_type=f32) + b4_r[...]
        mu = jnp.dot(h, mm_r[...], preferred_element_type=f32)
        hc = h - mu
        var = jnp.dot(hc * hc, mm_r[...], preferred_element_type=f32)
        xeo = hc * lax.rsqrt(var + 1e-5) * gv_r[...] + bv_r[...]
        o_ref[...] = xe + xeo
        y_ref[...] = jnp.dot(xeo, wyb_r[...], preferred_element_type=f32)

    data = pl.BlockSpec((block_rows, width), lambda i: (i, 0))
    mat = pl.BlockSpec((width, width), lambda i: (0, 0))
    vec = pl.BlockSpec((1, width), lambda i: (0, 0))
    out = jax.ShapeDtypeStruct((rows, width), jnp.float32)
    return pl.pallas_call(
        body,
        grid=grid,
        in_specs=[data, data, mat, vec, mat, vec, mat, vec, mat, vec,
                  vec, vec, mat, mat],
        out_specs=[data, data],
        out_shape=[out, out],
    )(g2, xe2, w1c, b1, w2, b2, w3, b3, w4, b4, gv, bv, mm, wyb)


def _node_mlp(x4, s4, w1a, b1, w2, b2, w3, b3, w4, b4, gv, bv, mm, block_rows):
    rows, width = x4.shape
    grid = (rows // block_rows,)

    def body(x_ref, s_ref, w1a_r, b1_r, w2_r, b2_r, w3_r, b3_r, w4_r, b4_r,
             gv_r, bv_r, mm_r, o_ref):
        f32 = jnp.float32
        x = x_ref[...]
        h = jnp.dot(x, w1a_r[...], preferred_element_type=f32) + s_ref[...]
        h = jnp.maximum(h + b1_r[...], 0.0)
        h = jnp.maximum(jnp.dot(h, w2_r[...], preferred_element_type=f32) + b2_r[...], 0.0)
        h = jnp.maximum(jnp.dot(h, w3_r[...], preferred_element_type=f32) + b3_r[...], 0.0)
        h = jnp.dot(h, w4_r[...], preferred_element_type=f32) + b4_r[...]
        mu = jnp.dot(h, mm_r[...], preferred_element_type=f32)
        hc = h - mu
        var = jnp.dot(hc * hc, mm_r[...], preferred_element_type=f32)
        o_ref[...] = x + hc * lax.rsqrt(var + 1e-5) * gv_r[...] + bv_r[...]

    data = pl.BlockSpec((block_rows, width), lambda i: (i, 0))
    mat = pl.BlockSpec((width, width), lambda i: (0, 0))
    vec = pl.BlockSpec((1, width), lambda i: (0, 0))
    out = jax.ShapeDtypeStruct((rows, width), jnp.float32)
    return pl.pallas_call(
        body,
        grid=grid,
        in_specs=[data, data, mat, vec, mat, vec, mat, vec, mat, vec,
                  vec, vec, mat],
        out_specs=data,
        out_shape=out,
    )(x4, s4, w1a, b1, w2, b2, w3, b3, w4, b4, gv, bv, mm)


def _sc_gather_sum(p, q, e0, e1):
    n, h = p.shape
    e = e0.shape[0]
    info = plsc.get_sparse_core_info()
    nw = info.num_cores * info.num_subcores
    epw = e // nw
    ch = 200
    nwin = epw // ch
    assert epw % ch == 0 and nwin % 2 == 0
    mesh = plsc.VectorSubcoreMesh(core_axis_name="c", subcore_axis_name="s")

    scratch = []
    for _ in range(2):
        scratch += [
            pltpu.VMEM((ch,), jnp.int32),
            pltpu.VMEM((ch,), jnp.int32),
            pltpu.VMEM((ch, h), jnp.float32),
            pltpu.VMEM((ch, h), jnp.float32),
            pltpu.VMEM((ch, h), jnp.float32),
            pltpu.SemaphoreType.DMA,
            pltpu.SemaphoreType.DMA,
        ]

    @functools.partial(
        pl.kernel,
        out_type=jax.ShapeDtypeStruct((e, h), jnp.float32),
        mesh=mesh,
        scratch_types=scratch,
        compiler_params=_SC_PARAMS,
    )
    def k(p_hbm, q_hbm, e0_hbm, e1_hbm, g_hbm, *scr):
        i0 = [scr[0], scr[7]]
        i1 = [scr[1], scr[8]]
        ba = [scr[2], scr[9]]
        bb = [scr[3], scr[10]]
        do = [scr[4], scr[11]]
        gs = [scr[5], scr[12]]
        st = [scr[6], scr[13]]
        wid = lax.axis_index("s") * info.num_cores + lax.axis_index("c")
        base = wid * epw

        def fetch(b, off):
            pltpu.sync_copy(e0_hbm.at[pl.ds(off, ch)], i0[b])
            pltpu.sync_copy(e1_hbm.at[pl.ds(off, ch)], i1[b])
            pltpu.async_copy(p_hbm.at[i0[b]], ba[b], gs[b])
            pltpu.async_copy(q_hbm.at[i1[b]], bb[b], gs[b])

        for b in range(2):
            fetch(b, base + b * ch)

        @pl.loop(0, nwin // 2)
        def _(gi):
            for b in range(2):
                off = base + (gi * 2 + b) * ch
                pltpu.make_async_copy(p_hbm.at[i0[b]], ba[b], gs[b]).wait()
                pltpu.make_async_copy(q_hbm.at[i1[b]], bb[b], gs[b]).wait()

                @pl.when(gi > 0)
                def _():
                    pltpu.make_async_copy(
                        do[b], g_hbm.at[pl.ds(off - 2 * ch, ch)], st[b]).wait()

                @pl.loop(0, ch)
                def _(r):
                    for half in range(h // LANES):
                        slc = (pl.ds(r, 1), pl.ds(half * LANES, LANES))
                        do[b].at[*slc][...] = (
                            ba[b].at[*slc][...] + bb[b].at[*slc][...])

                pltpu.async_copy(do[b], g_hbm.at[pl.ds(off, ch)], st[b])

                @pl.when(gi * 2 + b + 2 < nwin)
                def _():
                    fetch(b, off + 2 * ch)

        for b in range(2):
            pltpu.make_async_copy(
                do[b],
                g_hbm.at[pl.ds(base + (nwin - 2 + b) * ch, ch)],
                st[b]).wait()

    return k(p, q, e0, e1)


def _sc_scatter_add(ye, e0, e1, n):
    e, h = ye.shape
    info = plsc.get_sparse_core_info()
    nc, ns = info.num_cores, info.num_subcores
    hn = n // nc
    acc_rows = hn + ns
    eps = e // ns
    ch = 400
    nwin = eps // ch
    assert eps % ch == 0 and nwin % 2 == 0 and ch % LANES == 0
    zr = acc_rows // ns
    ops = hn // ns
    mesh = plsc.VectorSubcoreMesh(core_axis_name="c", subcore_axis_name="s")

    scratch = [pltpu.VMEM_SHARED((acc_rows, h), jnp.float32)]
    for _ in range(2):
        scratch += [
            pltpu.VMEM((ch, h), jnp.float32),
            pltpu.VMEM((ch,), jnp.int32),
            pltpu.VMEM((ch,), jnp.int32),
            pltpu.SemaphoreType.DMA,
        ]

    zrows = jnp.zeros((zr, h), jnp.float32)

    @functools.partial(
        pl.kernel,
        out_type=jax.ShapeDtypeStruct((n, h), jnp.float32),
        mesh=mesh,
        scratch_types=scratch,
        compiler_params=_SC_PARAMS,
    )
    def k(ye_hbm, e0_hbm, e1_hbm, z_hbm, s_hbm, *scr):
        acc = scr[0]
        yb = [scr[1], scr[5]]
        i0 = [scr[2], scr[6]]
        i1 = [scr[3], scr[7]]
        sc = [scr[4], scr[8]]
        c = lax.axis_index("c")
        sid = lax.axis_index("s")
        nbase = c * hn
        dummy = hn + sid

        pltpu.sync_copy(z_hbm, acc.at[pl.ds(sid * zr, zr)])
        plsc.subcore_barrier()

        ebase = sid * eps

        def load(b, off):
            pltpu.sync_copy(ye_hbm.at[pl.ds(off, ch)], yb[b])
            pltpu.sync_copy(e0_hbm.at[pl.ds(off, ch)], i0[b])
            pltpu.sync_copy(e1_hbm.at[pl.ds(off, ch)], i1[b])
            for iref in (i0[b], i1[b]):
                @pl.loop(0, ch // LANES)
                def _(kk):
                    s = pl.ds(kk * LANES, LANES)
                    v = iref.at[s][...]
                    inb = (v >= nbase) & (v < nbase + hn)
                    iref.at[s][...] = jnp.where(inb, v - nbase, dummy)

        def drain(b):
            pltpu.make_async_copy(yb[b], acc.at[i0[b]], sc[b]).wait()
            pltpu.make_async_copy(yb[b], acc.at[i1[b]], sc[b]).wait()

        load(0, ebase)

        @pl.loop(0, nwin // 2)
        def _(gi):
            for b in range(2):
                w0 = gi * 2 + b
                pltpu.async_copy(yb[b], acc.at[i0[b]], sc[b], add=True)
                pltpu.async_copy(yb[b], acc.at[i1[b]], sc[b], add=True)

                @pl.when(w0 + 1 < nwin)
                def _():
                    b2 = 1 - b

                    @pl.when(w0 >= 1)
                    def _():
                        drain(b2)

                    load(b2, ebase + (w0 + 1) * ch)

        for b in range(2):
            drain(b)

        plsc.subcore_barrier()
        pltpu.sync_copy(acc.at[pl.ds(sid * ops, ops)],
                        s_hbm.at[pl.ds(nbase + sid * ops, ops)])

    return k(ye, e0, e1, zrows)


def _bd(w, f):
    return jnp.kron(jnp.eye(f, dtype=w.dtype), w)


def _tile(v, f):
    return jnp.tile(v, f).reshape(1, f * v.shape[0])


def kernel(x_node, x_edge, edge_index, eb_W1, eb_b1, eb_W2, eb_b2, eb_W3,
           eb_b3, eb_W4, eb_b4, eb_g, eb_beta, nb_W1, nb_b1, nb_W2, nb_b2,
           nb_W3, nb_b3, nb_W4, nb_b4, nb_g, nb_beta):
    n = x_node.shape[0]
    e = x_edge.shape[0]
    e0 = edge_index[0]
    e1 = edge_index[1]

    fn, fe = 4, 4
    wn, we = fn * H, fe * H
    x4 = x_node.reshape(n // fn, wn)

    w1a4 = _bd(eb_W1[:H], fn)
    w1b4 = _bd(eb_W1[H:2 * H], fn)
    w1c8 = _bd(eb_W1[2 * H:], fe)
    eb_mats = [_bd(w, fe) for w in (eb_W2, eb_W3, eb_W4)]
    eb_vecs = [_tile(v, fe) for v in (eb_b1, eb_b2, eb_b3, eb_b4, eb_g, eb_beta)]
    mean8 = _bd(jnp.full((H, H), 1.0 / H, jnp.float32), fe)
    wyb8 = _bd(nb_W1[H:], fe)

    nb_mats = [_bd(w, fn) for w in (nb_W1[:H], nb_W2, nb_W3, nb_W4)]
    nb_vecs = [_tile(v, fn) for v in (nb_b1, nb_b2, nb_b3, nb_b4, nb_g, nb_beta)]
    mean4 = _bd(jnp.full((H, H), 1.0 / H, jnp.float32), fn)

    p4, q4 = _pq_tables(x4, w1a4, w1b4, block_rows=5000)
    p = p4.reshape(n, H)
    q = q4.reshape(n, H)

    g = _sc_gather_sum(p, q, e0, e1)

    out_e8, ye8 = _edge_mlp(
        g.reshape(e // fe, we), x_edge.reshape(e // fe, we),
        w1c8, eb_vecs[0], eb_mats[0], eb_vecs[1], eb_mats[1], eb_vecs[2],
        eb_mats[2], eb_vecs[3], eb_vecs[4], eb_vecs[5], mean8, wyb8,
        block_rows=2000)

    s = _sc_scatter_add(ye8.reshape(e, H), e0, e1, n)

    out_n4 = _node_mlp(
        x4, s.reshape(n // fn, wn),
        nb_mats[0], nb_vecs[0], nb_mats[1], nb_vecs[1], nb_mats[2],
        nb_vecs[2], nb_mats[3], nb_vecs[3], nb_vecs[4], nb_vecs[5], mean4,
        block_rows=1000)

    return out_n4.reshape(n, H), out_e8.reshape(e, H)
